# baseline (speedup 1.0000x reference)
import functools
import jax
import jax.numpy as jnp
from jax import lax
from jax.experimental import pallas as pl
from jax.experimental.pallas import tpu as pltpu
from jax.experimental.pallas import tpu_sc as plsc

_B, _P, _D = 4096, 100, 128
_NC, _NS, _L = 2, 16, 16
_NW = _NC * _NS
_PER_W = _B // _NW
_CB = 2
_NCHUNK = _PER_W // _CB
_NSLOT = 4
_VPR = _D // _L


def _sc_body(feat_hbm, pe_hbm, out_hbm, pe_v, b0, b1, b2, b3,
             si0, si1, si2, si3, so0, so1, so2, so3):
    bufs = (b0, b1, b2, b3)
    sin = (si0, si1, si2, si3)
    sout = (so0, so1, so2, so3)
    wid = lax.axis_index("s") * _NC + lax.axis_index("c")
    base = wid * _PER_W

    pltpu.sync_copy(pe_hbm, pe_v)

    def fetch(c, s):
        pltpu.async_copy(feat_hbm.at[pl.ds(base + c * _CB, _CB)], bufs[s], sin[s])

    def fetch_wait(s):
        pltpu.make_async_copy(feat_hbm.at[pl.ds(base, _CB)], bufs[s], sin[s]).wait()

    def flush(c, s):
        pltpu.async_copy(bufs[s], out_hbm.at[pl.ds(base + c * _CB, _CB)], sout[s])

    def flush_wait(s):
        pltpu.make_async_copy(bufs[s], out_hbm.at[pl.ds(base, _CB)], sout[s]).wait()

    fetch(0, 0)
    fetch(1, 1)

    def outer(c4, _):
        c0 = c4 * _NSLOT
        for b in range(_NSLOT):
            c = c0 + b
            sf = (b + 2) % _NSLOT

            @pl.when(c >= 2)
            def _(sf=sf):
                flush_wait(sf)

            @pl.when(c + 2 < _NCHUNK)
            def _(c=c, sf=sf):
                fetch(c + 2, sf)

            fetch_wait(b)
            buf = bufs[b]

            @plsc.parallel_loop(0, _P, 1, unroll=2)
            def _(p, _buf=buf):
                pevs = [pe_v[p, pl.ds(l * _L, _L)] for l in range(_VPR)]
                for i in range(_CB):
                    for l in range(_VPR):
                        plsc.addupdate(_buf.at[i, p, pl.ds(l * _L, _L)], pevs[l])

            flush(c, b)
        return ()

    lax.fori_loop(0, _NCHUNK // _NSLOT, outer, ())
    flush_wait(2)
    flush_wait(3)


def kernel(feat_tokens, pos_emb):
    mesh = plsc.VectorSubcoreMesh(core_axis_name="c", subcore_axis_name="s")
    run = functools.partial(
        pl.kernel,
        mesh=mesh,
        out_type=jax.ShapeDtypeStruct((_B, _P, _D), jnp.float32),
        scratch_types=(
            [pltpu.VMEM((_P, _D), jnp.float32)]
            + [pltpu.VMEM((_CB, _P, _D), jnp.float32) for _ in range(_NSLOT)]
            + [pltpu.SemaphoreType.DMA for _ in range(2 * _NSLOT)]
        ),
        compiler_params=pltpu.CompilerParams(use_tc_tiling_on_sc=True),
    )(_sc_body)
    return run(feat_tokens, pos_emb)


# speedup vs baseline: 3.3162x; 3.3162x over previous
import jax
import jax.numpy as jnp
from jax.experimental import pallas as pl


def _body(feat_ref, pe_ref, out_ref):
    p = pl.program_id(0)
    out_ref[...] = feat_ref[...] + pe_ref[p, :][None, None, :]


def kernel(feat_tokens, pos_emb):
    B, P, D = feat_tokens.shape
    feat_t = jnp.transpose(feat_tokens, (1, 0, 2))
    BB = 4096
    out_t = pl.pallas_call(
        _body,
        grid=(P, B // BB),
        in_specs=[
            pl.BlockSpec((1, BB, D), lambda p, i: (p, i, 0)),
            pl.BlockSpec((P, D), lambda p, i: (0, 0)),
        ],
        out_specs=pl.BlockSpec((1, BB, D), lambda p, i: (p, i, 0)),
        out_shape=jax.ShapeDtypeStruct((P, B, D), feat_tokens.dtype),
    )(feat_t, pos_emb)
    return jnp.transpose(out_t, (1, 0, 2))


# speedup vs baseline: 3.6792x; 1.1094x over previous
import jax
import jax.numpy as jnp
from jax.experimental import pallas as pl


_PBLK = 4


def _body(feat_ref, pe_ref, out_ref):
    p0 = pl.program_id(0) * _PBLK
    out_ref[...] = feat_ref[...] + pe_ref[pl.ds(p0, _PBLK), :][:, None, :]


def kernel(feat_tokens, pos_emb):
    B, P, D = feat_tokens.shape
    feat_t = jnp.transpose(feat_tokens, (1, 0, 2))
    BB = 4096
    out_t = pl.pallas_call(
        _body,
        grid=(P // _PBLK, B // BB),
        in_specs=[
            pl.BlockSpec((_PBLK, BB, D), lambda p, i: (p, i, 0)),
            pl.BlockSpec((P, D), lambda p, i: (0, 0)),
        ],
        out_specs=pl.BlockSpec((_PBLK, BB, D), lambda p, i: (p, i, 0)),
        out_shape=jax.ShapeDtypeStruct((P, B, D), feat_tokens.dtype),
    )(feat_t, pos_emb)
    return jnp.transpose(out_t, (1, 0, 2))
